# baseline (speedup 1.0000x reference)
import functools

import jax
import jax.numpy as jnp
from jax import lax
from jax.experimental import pallas as pl
from jax.experimental.pallas import tpu as pltpu
from jax.experimental.pallas import tpu_sc as plsc

_EMBED = 64
_CHUNK = 128


@functools.lru_cache(maxsize=None)
def _make_gather(V, D, B):
    info = plsc.get_sparse_core_info()
    NC, NS = info.num_cores, info.num_subcores
    NW = NC * NS
    assert B % (NW * _CHUNK) == 0
    b_per_w = B // NW
    n_chunks = b_per_w // _CHUNK
    mesh = plsc.VectorSubcoreMesh(core_axis_name="c", subcore_axis_name="s")

    @functools.partial(
        pl.kernel,
        mesh=mesh,
        out_type=jax.ShapeDtypeStruct((B, D), jnp.float32),
        scratch_types=[
            pltpu.VMEM((n_chunks, _CHUNK), jnp.int32),
            pltpu.VMEM((_CHUNK, D), jnp.float32),
            pltpu.SemaphoreType.DMA,
        ],
        compiler_params=pltpu.CompilerParams(use_tc_tiling_on_sc=False),
    )
    def gather_kernel(idx_hbm, table_hbm, out_hbm, idx_v, rows_v, sem):
        wid = lax.axis_index("s") * NC + lax.axis_index("c")
        pltpu.sync_copy(idx_hbm.at[wid], idx_v)

        def body(j, carry):
            pltpu.async_copy(table_hbm.at[idx_v.at[j]], rows_v, sem).wait()
            pltpu.sync_copy(
                rows_v, out_hbm.at[pl.ds(wid * b_per_w + j * _CHUNK, _CHUNK)]
            )
            return carry

        lax.fori_loop(0, n_chunks, body, 0)

    return gather_kernel


def kernel(inputs, embeddings):
    V, D = embeddings.shape
    B = inputs.size
    info = plsc.get_sparse_core_info()
    NW = info.num_cores * info.num_subcores
    idx = inputs.reshape(NW, B // (NW * _CHUNK), _CHUNK).astype(jnp.int32)
    out = _make_gather(V, D, B)(idx, embeddings)
    return out.reshape(inputs.shape + (D,))


# speedup vs baseline: 1.0449x; 1.0449x over previous
import functools

import jax
import jax.numpy as jnp
from jax import lax
from jax.experimental import pallas as pl
from jax.experimental.pallas import tpu as pltpu
from jax.experimental.pallas import tpu_sc as plsc

_EMBED = 64
_CHUNK = 128


@functools.lru_cache(maxsize=None)
def _make_gather(V, D, B):
    info = plsc.get_sparse_core_info()
    NC, NS = info.num_cores, info.num_subcores
    NW = NC * NS
    assert B % (NW * _CHUNK) == 0
    b_per_w = B // NW
    n_chunks = b_per_w // _CHUNK
    mesh = plsc.VectorSubcoreMesh(core_axis_name="c", subcore_axis_name="s")

    NBUF = 5
    assert n_chunks % NBUF == 0
    n_groups = n_chunks // NBUF

    @functools.partial(
        pl.kernel,
        mesh=mesh,
        out_type=jax.ShapeDtypeStruct((B, D), jnp.float32),
        scratch_types=[
            pltpu.VMEM((n_chunks, _CHUNK), jnp.int32),
            pltpu.VMEM((NBUF, _CHUNK, D), jnp.float32),
            [pltpu.SemaphoreType.DMA] * NBUF,
            [pltpu.SemaphoreType.DMA] * NBUF,
        ],
        compiler_params=pltpu.CompilerParams(use_tc_tiling_on_sc=False),
    )
    def gather_kernel(idx_hbm, table_hbm, out_hbm, idx_v, rows_v, gsems, osems):
        wid = lax.axis_index("s") * NC + lax.axis_index("c")
        out_base = wid * b_per_w
        pltpu.sync_copy(idx_hbm.at[wid], idx_v)

        for b in range(NBUF):
            pltpu.async_copy(table_hbm.at[idx_v.at[b]], rows_v.at[b], gsems[b])

        def body(g, carry):
            j0 = g * NBUF
            for b in range(NBUF):
                pltpu.make_async_copy(
                    table_hbm.at[idx_v.at[b]], rows_v.at[b], gsems[b]
                ).wait()
                pltpu.async_copy(
                    rows_v.at[b],
                    out_hbm.at[pl.ds(out_base + (j0 + b) * _CHUNK, _CHUNK)],
                    osems[b],
                )
                pltpu.make_async_copy(
                    rows_v.at[b],
                    out_hbm.at[pl.ds(out_base + (j0 + b) * _CHUNK, _CHUNK)],
                    osems[b],
                ).wait()

                @pl.when(g < n_groups - 1)
                def _():
                    pltpu.async_copy(
                        table_hbm.at[idx_v.at[j0 + NBUF + b]],
                        rows_v.at[b],
                        gsems[b],
                    )

            return carry

        lax.fori_loop(0, n_groups, body, 0)

    return gather_kernel


def kernel(inputs, embeddings):
    V, D = embeddings.shape
    B = inputs.size
    info = plsc.get_sparse_core_info()
    NW = info.num_cores * info.num_subcores
    idx = inputs.reshape(NW, B // (NW * _CHUNK), _CHUNK).astype(jnp.int32)
    out = _make_gather(V, D, B)(idx, embeddings)
    return out.reshape(inputs.shape + (D,))
